# baseline (speedup 1.0000x reference)
import functools

import jax
import jax.numpy as jnp
from jax import lax
from jax.experimental import pallas as pl
from jax.experimental.pallas import tpu as pltpu
from jax.experimental.pallas import tpu_sc as plsc

LANES = 16
NUM_CORES = 2
NUM_SUBCORES = 16
NUM_WORKERS = NUM_CORES * NUM_SUBCORES
CHUNK = 64
N_SLOT = 4
N_OSLOT = 3


def _tec_body(heads_hbm, rel_hbm, tails_hbm, entity_hbm, relw_hbm, out_hbm,
              idx_h, idx_r, idx_t, gbufs, obufs, gsems, osems):
    batch = out_hbm.shape[0]
    embed = out_hbm.shape[1]
    b_per_w = batch // NUM_WORKERS
    n_chunks = b_per_w // CHUNK
    wid = lax.axis_index("s") * NUM_CORES + lax.axis_index("c")
    base = wid * b_per_w

    pltpu.sync_copy(heads_hbm.at[pl.ds(base, b_per_w)], idx_h)
    pltpu.sync_copy(rel_hbm.at[pl.ds(base, b_per_w)], idx_r)
    pltpu.sync_copy(tails_hbm.at[pl.ds(base, b_per_w)], idx_t)

    def start_gather(c):
        h, r, t = gbufs[c % N_SLOT]
        gs = gsems[c % N_SLOT]
        s = pl.ds(c * CHUNK, CHUNK)
        return (pltpu.async_copy(entity_hbm.at[idx_h.at[s]], h, gs),
                pltpu.async_copy(relw_hbm.at[idx_r.at[s]], r, gs),
                pltpu.async_copy(entity_hbm.at[idx_t.at[s]], t, gs))

    gathers = [None] * n_chunks
    writes = [None] * n_chunks
    for c in range(min(N_SLOT, n_chunks)):
        gathers[c] = start_gather(c)

    for c in range(n_chunks):
        oslot = c % N_OSLOT
        for cp in gathers[c]:
            cp.wait()
        if c >= N_OSLOT:
            writes[c - N_OSLOT].wait()
        h, r, t = gbufs[c % N_SLOT]
        o = obufs[oslot]

        @plsc.parallel_loop(0, CHUNK, 1, unroll=4)
        def _compute(j):
            for k in range(embed // LANES):
                s = pl.ds(k * LANES, LANES)
                o[j, s] = h[j, s] + r[j, s] - t[j, s]

        writes[c] = pltpu.async_copy(
            o, out_hbm.at[pl.ds(base + c * CHUNK, CHUNK)], osems[oslot])
        if c + N_SLOT < n_chunks:
            gathers[c + N_SLOT] = start_gather(c + N_SLOT)

    for c in range(max(0, n_chunks - N_OSLOT), n_chunks):
        writes[c].wait()


def _body_wrapper(heads_hbm, rel_hbm, tails_hbm, entity_hbm, relw_hbm,
                  out_hbm, idx_h, idx_r, idx_t, *bufs_and_sems):
    n, m = N_SLOT, N_OSLOT
    gbufs = tuple((bufs_and_sems[3 * i], bufs_and_sems[3 * i + 1],
                   bufs_and_sems[3 * i + 2]) for i in range(n))
    obufs = tuple(bufs_and_sems[3 * n:3 * n + m])
    gsems = tuple(bufs_and_sems[3 * n + m:4 * n + m])
    osems = tuple(bufs_and_sems[4 * n + m:4 * n + 2 * m])
    _tec_body(heads_hbm, rel_hbm, tails_hbm, entity_hbm, relw_hbm, out_hbm,
              idx_h, idx_r, idx_t, gbufs, obufs, gsems, osems)


def kernel(heads, relations, tails, entity_weight, rel_weight):
    batch = heads.shape[0]
    embed = entity_weight.shape[1]
    b_per_w = batch // NUM_WORKERS
    heads = heads.astype(jnp.int32)
    relations = relations.astype(jnp.int32)
    tails = tails.astype(jnp.int32)

    mesh = plsc.VectorSubcoreMesh(core_axis_name="c", subcore_axis_name="s")
    scratch = [pltpu.VMEM((b_per_w,), jnp.int32)] * 3
    scratch += [pltpu.VMEM((CHUNK, embed), jnp.float32)] * (3 * N_SLOT)
    scratch += [pltpu.VMEM((CHUNK, embed), jnp.float32)] * N_OSLOT
    scratch += [pltpu.SemaphoreType.DMA] * N_SLOT
    scratch += [pltpu.SemaphoreType.DMA] * N_OSLOT
    run = functools.partial(
        pl.kernel,
        mesh=mesh,
        out_type=jax.ShapeDtypeStruct((batch, embed), jnp.float32),
        scratch_types=scratch,
    )(_body_wrapper)
    return run(heads, relations, tails, entity_weight, rel_weight)


# speedup vs baseline: 1.0741x; 1.0741x over previous
import functools

import jax
import jax.numpy as jnp
from jax import lax
from jax.experimental import pallas as pl
from jax.experimental.pallas import tpu as pltpu
from jax.experimental.pallas import tpu_sc as plsc

LANES = 16
NUM_CORES = 2
NUM_SUBCORES = 16
NUM_WORKERS = NUM_CORES * NUM_SUBCORES
CHUNK = 64
N_SLOT = 3


def _tec_body(heads_hbm, rel_hbm, tails_hbm, entity_hbm, relw_hbm, out_hbm,
              idx_h, idx_r, idx_t, hrbufs, tbufs, obufs, gsems, rsems, osems):
    batch = out_hbm.shape[0]
    embed = out_hbm.shape[1]
    b_per_w = batch // NUM_WORKERS
    n_chunks = b_per_w // CHUNK
    wid = lax.axis_index("s") * NUM_CORES + lax.axis_index("c")
    base = wid * b_per_w

    pltpu.sync_copy(heads_hbm.at[pl.ds(base, b_per_w)], idx_h)
    pltpu.sync_copy(rel_hbm.at[pl.ds(base, b_per_w)], idx_r)
    pltpu.sync_copy(tails_hbm.at[pl.ds(base, b_per_w)], idx_t)

    def start_ht(c):
        slot = c % N_SLOT
        s = pl.ds(c * CHUNK, CHUNK)
        return (pltpu.async_copy(entity_hbm.at[idx_h.at[s]], hrbufs[slot],
                                 gsems[slot]),
                pltpu.async_copy(entity_hbm.at[idx_t.at[s]], tbufs[slot],
                                 gsems[slot]))

    def start_radd(c):
        slot = c % N_SLOT
        s = pl.ds(c * CHUNK, CHUNK)
        return pltpu.async_copy(relw_hbm.at[idx_r.at[s]], hrbufs[slot],
                                rsems[slot], add=True)

    gathers = [None] * n_chunks
    radds = [None] * n_chunks
    writes = [None] * n_chunks
    for c in range(min(N_SLOT, n_chunks)):
        gathers[c] = start_ht(c)
    gathers[0][0].wait()
    radds[0] = start_radd(0)

    for c in range(n_chunks):
        slot = c % N_SLOT
        if c + 1 < n_chunks:
            gathers[c + 1][0].wait()
            radds[c + 1] = start_radd(c + 1)
        radds[c].wait()
        gathers[c][1].wait()
        if c >= N_SLOT:
            writes[c - N_SLOT].wait()
        hr = hrbufs[slot]
        t = tbufs[slot]
        o = obufs[slot]

        @plsc.parallel_loop(0, CHUNK, 1, unroll=2)
        def _compute(j):
            for k in range(embed // LANES):
                s = pl.ds(k * LANES, LANES)
                o[j, s] = hr[j, s] - t[j, s]

        writes[c] = pltpu.async_copy(
            o, out_hbm.at[pl.ds(base + c * CHUNK, CHUNK)], osems[slot])
        if c + N_SLOT < n_chunks:
            gathers[c + N_SLOT] = start_ht(c + N_SLOT)

    for c in range(max(0, n_chunks - N_SLOT), n_chunks):
        writes[c].wait()


def _body_wrapper(heads_hbm, rel_hbm, tails_hbm, entity_hbm, relw_hbm,
                  out_hbm, idx_h, idx_r, idx_t, *bufs_and_sems):
    n = N_SLOT
    hrbufs = tuple(bufs_and_sems[0:n])
    tbufs = tuple(bufs_and_sems[n:2 * n])
    obufs = tuple(bufs_and_sems[2 * n:3 * n])
    gsems = tuple(bufs_and_sems[3 * n:4 * n])
    rsems = tuple(bufs_and_sems[4 * n:5 * n])
    osems = tuple(bufs_and_sems[5 * n:6 * n])
    _tec_body(heads_hbm, rel_hbm, tails_hbm, entity_hbm, relw_hbm, out_hbm,
              idx_h, idx_r, idx_t, hrbufs, tbufs, obufs, gsems, rsems, osems)


def kernel(heads, relations, tails, entity_weight, rel_weight):
    batch = heads.shape[0]
    embed = entity_weight.shape[1]
    b_per_w = batch // NUM_WORKERS
    heads = heads.astype(jnp.int32)
    relations = relations.astype(jnp.int32)
    tails = tails.astype(jnp.int32)

    mesh = plsc.VectorSubcoreMesh(core_axis_name="c", subcore_axis_name="s")
    scratch = [pltpu.VMEM((b_per_w,), jnp.int32)] * 3
    scratch += [pltpu.VMEM((CHUNK, embed), jnp.float32)] * (3 * N_SLOT)
    scratch += [pltpu.SemaphoreType.DMA] * (3 * N_SLOT)
    run = functools.partial(
        pl.kernel,
        mesh=mesh,
        out_type=jax.ShapeDtypeStruct((batch, embed), jnp.float32),
        scratch_types=scratch,
    )(_body_wrapper)
    return run(heads, relations, tails, entity_weight, rel_weight)


# speedup vs baseline: 1.1628x; 1.0826x over previous
import functools

import jax
import jax.numpy as jnp
from jax import lax
from jax.experimental import pallas as pl
from jax.experimental.pallas import tpu as pltpu
from jax.experimental.pallas import tpu_sc as plsc

LANES = 16
NUM_CORES = 2
NUM_SUBCORES = 16
NUM_WORKERS = NUM_CORES * NUM_SUBCORES
CHUNK = 64
N_SLOT = 3


def _tec_body(heads_hbm, rel_hbm, tails_hbm, entity_hbm, relw_hbm, out_hbm,
              idx_h, idx_r, idx_t, rel_sh, hrbufs, tbufs, obufs,
              gsems, rsems, osems):
    batch = out_hbm.shape[0]
    embed = out_hbm.shape[1]
    b_per_w = batch // NUM_WORKERS
    n_chunks = b_per_w // CHUNK
    sid = lax.axis_index("s")
    wid = sid * NUM_CORES + lax.axis_index("c")
    base = wid * b_per_w

    pltpu.sync_copy(heads_hbm.at[pl.ds(base, b_per_w)], idx_h)
    pltpu.sync_copy(rel_hbm.at[pl.ds(base, b_per_w)], idx_r)
    pltpu.sync_copy(tails_hbm.at[pl.ds(base, b_per_w)], idx_t)

    def start_ht(c):
        slot = c % N_SLOT
        s = pl.ds(c * CHUNK, CHUNK)
        return (pltpu.async_copy(entity_hbm.at[idx_h.at[s]], hrbufs[slot],
                                 gsems[slot]),
                pltpu.async_copy(entity_hbm.at[idx_t.at[s]], tbufs[slot],
                                 gsems[slot]))

    def start_radd(c):
        slot = c % N_SLOT
        s = pl.ds(c * CHUNK, CHUNK)
        return pltpu.async_copy(rel_sh.at[idx_r.at[s]], hrbufs[slot],
                                rsems[slot], add=True)

    gathers = [None] * n_chunks
    radds = [None] * n_chunks
    writes = [None] * n_chunks
    for c in range(min(N_SLOT, n_chunks)):
        gathers[c] = start_ht(c)
    @pl.when(sid == 0)
    def _stage_rel():
        pltpu.sync_copy(relw_hbm, rel_sh)
    plsc.subcore_barrier()
    gathers[0][0].wait()
    radds[0] = start_radd(0)

    for c in range(n_chunks):
        slot = c % N_SLOT
        if c + 1 < n_chunks:
            gathers[c + 1][0].wait()
            radds[c + 1] = start_radd(c + 1)
        radds[c].wait()
        gathers[c][1].wait()
        if c >= N_SLOT:
            writes[c - N_SLOT].wait()
        hr = hrbufs[slot]
        t = tbufs[slot]
        o = obufs[slot]

        @plsc.parallel_loop(0, CHUNK, 1, unroll=2)
        def _compute(j):
            for k in range(embed // LANES):
                s = pl.ds(k * LANES, LANES)
                o[j, s] = hr[j, s] - t[j, s]

        writes[c] = pltpu.async_copy(
            o, out_hbm.at[pl.ds(base + c * CHUNK, CHUNK)], osems[slot])
        if c + N_SLOT < n_chunks:
            gathers[c + N_SLOT] = start_ht(c + N_SLOT)

    for c in range(max(0, n_chunks - N_SLOT), n_chunks):
        writes[c].wait()


def _body_wrapper(heads_hbm, rel_hbm, tails_hbm, entity_hbm, relw_hbm,
                  out_hbm, idx_h, idx_r, idx_t, *bufs_and_sems):
    n = N_SLOT
    rel_sh = bufs_and_sems[-1]
    bufs_and_sems = bufs_and_sems[:-1]
    hrbufs = tuple(bufs_and_sems[0:n])
    tbufs = tuple(bufs_and_sems[n:2 * n])
    obufs = tuple(bufs_and_sems[2 * n:3 * n])
    gsems = tuple(bufs_and_sems[3 * n:4 * n])
    rsems = tuple(bufs_and_sems[4 * n:5 * n])
    osems = tuple(bufs_and_sems[5 * n:6 * n])
    _tec_body(heads_hbm, rel_hbm, tails_hbm, entity_hbm, relw_hbm, out_hbm,
              idx_h, idx_r, idx_t, rel_sh, hrbufs, tbufs, obufs,
              gsems, rsems, osems)


def kernel(heads, relations, tails, entity_weight, rel_weight):
    batch = heads.shape[0]
    embed = entity_weight.shape[1]
    b_per_w = batch // NUM_WORKERS
    heads = heads.astype(jnp.int32)
    relations = relations.astype(jnp.int32)
    tails = tails.astype(jnp.int32)

    mesh = plsc.VectorSubcoreMesh(core_axis_name="c", subcore_axis_name="s")
    scratch = [pltpu.VMEM((b_per_w,), jnp.int32)] * 3
    scratch += [pltpu.VMEM((CHUNK, embed), jnp.float32)] * (3 * N_SLOT)
    scratch += [pltpu.SemaphoreType.DMA] * (3 * N_SLOT)
    scratch += [pltpu.VMEM_SHARED(rel_weight.shape, jnp.float32)]
    run = functools.partial(
        pl.kernel,
        mesh=mesh,
        out_type=jax.ShapeDtypeStruct((batch, embed), jnp.float32),
        scratch_types=scratch,
    )(_body_wrapper)
    return run(heads, relations, tails, entity_weight, rel_weight)
